# baseline (speedup 1.0000x reference)
import functools

import jax
import jax.numpy as jnp
from jax import lax
from jax.experimental import pallas as pl
from jax.experimental.pallas import tpu as pltpu
from jax.experimental.pallas import tpu_sc as plsc

N = 2048
E = 8192
M = 128
N_DIM = 64
E_DIM = 16
H = 128
HE = 64
L = 3
C = 128

NC = 2
NS = 16
NW = NC * NS
EPT = E // NW
CH = 64
NCH = EPT // CH
TW = 256
ACOL = H + HE
NROWS = N // NS
DUMP = N
NACC = N + 16


def _prep_body(nf, ef, us64, vs64, Wn, bn, We, be, WU, WV, WE, bcat,
               h_out, tabU_out, tabV_out, et_out, vsm_out):
    dot = functools.partial(jnp.dot, preferred_element_type=jnp.float32)
    h = jax.nn.leaky_relu(dot(nf[...], Wn[...]) + bn[...])
    e = jax.nn.leaky_relu(dot(ef[...], We[...]) + be[...])
    h_out[...] = h
    tabU_out[...] = dot(h, WU[...])
    tabV_out[...] = dot(h, WV[...])
    et_out[...] = dot(e, WE[...]) + bcat[...]
    vsm_out[...] = jnp.where(us64[...] == vs64[...], DUMP, vs64[...])


def _gru_body(nums, dens, h, nesum, Wih_i, bih_i, Whh_i, bhh_i,
              WU, WV, WE, bcat, relu_flag, last,
              h_out, tabU_out, tabV_out, et_out):
    dot = functools.partial(jnp.dot, preferred_element_type=jnp.float32)
    s = nums[0] + nums[1]
    den = jnp.maximum(dens[0, :, :1] + dens[1, :, :1], 1e-30)
    ctx = s / den
    gi = dot(ctx, Wih_i[...]) + bih_i[...]
    gh = dot(h[...], Whh_i[...]) + bhh_i[...]
    r = jax.nn.sigmoid(gi[:, :H] + gh[:, :H])
    z = jax.nn.sigmoid(gi[:, H:2 * H] + gh[:, H:2 * H])
    n = jnp.tanh(gi[:, 2 * H:] + r * gh[:, 2 * H:])
    new_h = (1.0 - z) * n + z * h[...]
    if relu_flag:
        new_h = jax.nn.relu(new_h)
    h_out[...] = new_h
    if not last:
        tabU_out[...] = dot(new_h, WU[...])
        tabV_out[...] = dot(new_h, WV[...])
        ne = jax.nn.leaky_relu(nesum[:, :HE])
        et_out[...] = dot(ne, WE[...]) + bcat[...]


def _readout_body(h, mask, mat, Wra, bra, Wro, bro, ro_out, a_out):
    hh = h[...]
    t = jnp.tanh(jnp.dot(hh, Wro[...], preferred_element_type=jnp.float32) + bro[...])
    alT = lax.dot_general(Wra[...], hh, (((0,), (1,)), ((), ())),
                          preferred_element_type=jnp.float32)
    alT = jax.nn.leaky_relu(alT + bra[...])
    logits = mask[...] + alT
    rmax = jnp.max(logits, axis=1, keepdims=True)
    ex = jnp.exp(logits - rmax)
    a = ex / jnp.sum(ex, axis=1, keepdims=True) * mat[...]
    a_out[...] = a
    ro_out[...] = jnp.dot(a, t, preferred_element_type=jnp.float32)


def _tc_prep(nf, ef, us64, vs64, Wn, bn, We, be, WU, WV, WE, bcat):
    return pl.pallas_call(
        _prep_body,
        out_shape=[jax.ShapeDtypeStruct((N, H), jnp.float32),
                   jax.ShapeDtypeStruct((N, TW), jnp.float32),
                   jax.ShapeDtypeStruct((N, TW), jnp.float32),
                   jax.ShapeDtypeStruct((E, TW), jnp.float32),
                   jax.ShapeDtypeStruct((E // CH, CH), jnp.int32)],
    )(nf, ef, us64, vs64, Wn, bn, We, be, WU, WV, WE, bcat)


def _tc_gru(nums, dens, h, nesum, Wih_i, bih_i, Whh_i, bhh_i,
            WU, WV, WE, bcat, relu_flag, last):
    def wrapped(a1, a2, a3, a4, a5, a6, a7, a8, a9, a10, a11, a12, *outs):
        _gru_body(a1, a2, a3, a4, a5, a6, a7, a8, a9, a10, a11, a12,
                  relu_flag, last, *outs)

    out_shape = [jax.ShapeDtypeStruct((N, H), jnp.float32),
                 jax.ShapeDtypeStruct((N, TW), jnp.float32),
                 jax.ShapeDtypeStruct((N, TW), jnp.float32),
                 jax.ShapeDtypeStruct((E, TW), jnp.float32)]
    if last:
        out_shape = out_shape[:1]

        def wrapped(a1, a2, a3, a4, a5, a6, a7, a8, a9, a10, a11, a12, o1):
            _gru_body(a1, a2, a3, a4, a5, a6, a7, a8, a9, a10, a11, a12,
                      relu_flag, last, o1, None, None, None)

    return pl.pallas_call(
        wrapped,
        out_shape=out_shape,
    )(nums, dens, h, nesum, Wih_i, bih_i, Whh_i, bhh_i, WU, WV, WE, bcat)


def _tc_readout(h, mask, mat, Wra, bra, Wro, bro):
    return pl.pallas_call(
        _readout_body,
        out_shape=[jax.ShapeDtypeStruct((M, H), jnp.float32),
                   jax.ShapeDtypeStruct((M, N), jnp.float32)],
    )(h, mask, mat, Wra, bra, Wro, bro)


def _sc_layer(tabU, tabV, et, us64, vs64, vsm64, zer):
    mesh = plsc.VectorSubcoreMesh(core_axis_name="c", subcore_axis_name="s")

    @functools.partial(
        pl.kernel,
        out_type=[jax.ShapeDtypeStruct((2, N, H), jnp.float32),
                  jax.ShapeDtypeStruct((2, N, H), jnp.float32),
                  jax.ShapeDtypeStruct((E, H), jnp.float32)],
        mesh=mesh,
        scratch_types=[pltpu.VMEM((NCH, CH), jnp.int32),
                       pltpu.VMEM((NCH, CH), jnp.int32),
                       pltpu.VMEM((NCH, CH), jnp.int32),
                       pltpu.VMEM((CH, TW), jnp.float32),
                       pltpu.VMEM((CH, TW), jnp.float32),
                       pltpu.VMEM((CH, TW), jnp.float32),
                       pltpu.VMEM((CH, H), jnp.float32),
                       pltpu.VMEM((CH, H), jnp.float32),
                       pltpu.VMEM((CH, H), jnp.float32),
                       pltpu.VMEM((CH,), jnp.float32),
                       pltpu.VMEM_SHARED((NACC, H), jnp.float32),
                       pltpu.VMEM_SHARED((NACC, H), jnp.float32)],
        compiler_params=pltpu.CompilerParams(needs_layout_passes=False),
    )
    def k(tabU_h, tabV_h, et_h, us_h, vs_h, vsm_h, zer_h,
          nums_out, dens_out, nesum_out,
          idxu, idxv, idxvm, gA, gB, etb, wmb, wdb, nb, wbuf, acc_n, acc_d):
        cid = lax.axis_index("c")
        sid = lax.axis_index("s")
        wid = sid * NC + cid
        base = wid * EPT
        pltpu.sync_copy(zer_h.at[pl.ds(sid * NROWS, NROWS)],
                        acc_n.at[pl.ds(sid * NROWS, NROWS)])
        pltpu.sync_copy(zer_h.at[pl.ds(sid * NROWS, NROWS)],
                        acc_d.at[pl.ds(sid * NROWS, NROWS)])
        pltpu.sync_copy(us_h.at[pl.ds(wid * NCH, NCH)], idxu)
        pltpu.sync_copy(vs_h.at[pl.ds(wid * NCH, NCH)], idxv)
        pltpu.sync_copy(vsm_h.at[pl.ds(wid * NCH, NCH)], idxvm)
        pltpu.sync_copy(zer_h.at[pl.ds(0, CH)], wdb)
        pltpu.sync_copy(zer_h.at[pl.ds(0, CH)], nb)
        plsc.subcore_barrier()
        for c in range(NCH):
            pltpu.sync_copy(tabU_h.at[idxu.at[c]], gA)
            pltpu.sync_copy(tabV_h.at[idxv.at[c]], gB)
            pltpu.sync_copy(et_h.at[pl.ds(base + c * CH, CH)], etb)
            for g in range(CH // 16):
                rows = lax.broadcasted_iota(jnp.int32, (16,), 0) + g * 16
                cols = jnp.full((16,), ACOL, jnp.int32)
                a = (plsc.load_gather(gA, [rows, cols])
                     + plsc.load_gather(gB, [rows, cols])
                     + plsc.load_gather(etb, [rows, cols]))
                a = jnp.maximum(a, 0.01 * a)
                a = jnp.minimum(a, 80.0)
                w16 = jnp.exp(a)
                wbuf[pl.ds(g * 16, 16)] = w16
                plsc.store_scatter(wdb, [rows, jnp.zeros((16,), jnp.int32)],
                                   w16)

            def mbody(e, carry):
                wv = plsc.load_gather(wbuf, [jnp.full((16,), e, jnp.int32)])
                for k8 in range(H // 16):
                    sl = pl.ds(k8 * 16, 16)
                    x = gA[e, sl] + gB[e, sl] + etb[e, sl]
                    wmb[e, sl] = jnp.maximum(x, 0.0) * wv
                for k4 in range(HE // 16):
                    sl = pl.ds(H + k4 * 16, 16)
                    y = gA[e, sl] + gB[e, sl] + etb[e, sl]
                    nb[e, pl.ds(k4 * 16, 16)] = y
                return carry
            lax.fori_loop(0, CH, mbody, 0)
            pltpu.sync_copy(wmb, acc_n.at[idxu.at[c]], add=True)
            pltpu.sync_copy(wmb, acc_n.at[idxvm.at[c]], add=True)
            pltpu.sync_copy(wdb, acc_d.at[idxu.at[c]], add=True)
            pltpu.sync_copy(wdb, acc_d.at[idxvm.at[c]], add=True)
            pltpu.sync_copy(nb, nesum_out.at[pl.ds(base + c * CH, CH)])
        plsc.subcore_barrier()
        pltpu.sync_copy(acc_n.at[pl.ds(sid * NROWS, NROWS)],
                        nums_out.at[cid, pl.ds(sid * NROWS, NROWS)])
        pltpu.sync_copy(acc_d.at[pl.ds(sid * NROWS, NROWS)],
                        dens_out.at[cid, pl.ds(sid * NROWS, NROWS)])

    return k(tabU, tabV, et, us64, vs64, vsm64, zer)


def _pack_w(Wm_i, Wen_i, Wa_i):
    pad = jnp.zeros((Wm_i.shape[0], TW - ACOL - 1), jnp.float32)
    return jnp.concatenate([Wm_i, Wen_i, Wa_i, pad], axis=1)


def kernel(node_features, edge_features, us, vs, mol_node_matrix, mol_node_mask,
           node_edge_matrix, node_edge_mask, global_mask, W_n, b_n, W_e, b_e,
           Wm, bm, Wa, ba, Wen, ben, Wih, bih, Whh, bhh, Wra, bra, Wro, bro):
    us64 = us.reshape(E // CH, CH)
    vs64 = vs.reshape(E // CH, CH)
    zer = jnp.zeros((N, H), jnp.float32)

    D0, D1 = H, H + HE
    WUs = [_pack_w(Wm[i][:D0], Wen[i][:D0], Wa[i][:D0]) for i in range(L)]
    WVs = [_pack_w(Wm[i][D1:], Wen[i][D1:], Wa[i][D1:]) for i in range(L)]
    WEs = [_pack_w(Wm[i][D0:D1], Wen[i][D0:D1], Wa[i][D0:D1]) for i in range(L)]
    bcats = [jnp.concatenate([bm[i], ben[i], ba[i],
                              jnp.zeros((TW - ACOL - 1,), jnp.float32)]
                             ).reshape(1, TW) for i in range(L)]

    h, tabU, tabV, et, vsm64 = _tc_prep(
        node_features, edge_features, us64, vs64,
        W_n, b_n.reshape(1, H), W_e, b_e.reshape(1, HE),
        WUs[0], WVs[0], WEs[0], bcats[0])

    for i in range(L):
        nums, dens, nesum = _sc_layer(tabU, tabV, et, us64, vs64, vsm64, zer)
        last = (i == L - 1)
        if last:
            WU1 = WV1 = jnp.zeros((H, TW), jnp.float32)
            WE1 = jnp.zeros((HE, TW), jnp.float32)
            bc1 = jnp.zeros((1, TW), jnp.float32)
        else:
            WU1, WV1, WE1, bc1 = WUs[i + 1], WVs[i + 1], WEs[i + 1], bcats[i + 1]
        nesum_in = jnp.zeros((8, H), jnp.float32) if last else nesum
        outs = _tc_gru(nums, dens, h, nesum_in, Wih[i], bih[i].reshape(1, 3 * H),
                       Whh[i], bhh[i].reshape(1, 3 * H),
                       WU1, WV1, WE1, bc1,
                       relu_flag=(not last), last=last)
        if last:
            h = outs[0]
        else:
            h, tabU, tabV, et = outs

    readout, a = _tc_readout(h, mol_node_mask, mol_node_matrix,
                             Wra, bra.reshape(1, 1), Wro, bro.reshape(1, H))
    return readout, a


# speedup vs baseline: 1.6283x; 1.6283x over previous
import functools

import jax
import jax.numpy as jnp
from jax import lax
from jax.experimental import pallas as pl
from jax.experimental.pallas import tpu as pltpu
from jax.experimental.pallas import tpu_sc as plsc

N = 2048
E = 8192
M = 128
N_DIM = 64
E_DIM = 16
H = 128
HE = 64
L = 3
C = 128

NC = 2
NS = 16
NW = NC * NS
EPT = E // NW
IPW = EPT // 128
WCOL = 128
NROWS = N // NS
DUMP = N
NACC = N + 16


def _proj_body(nf, us2, vs2, Wn, bn, h_out, vsm_out):
    h_out[...] = jax.nn.leaky_relu(
        jnp.dot(nf[...], Wn[...], preferred_element_type=jnp.float32) + bn[...])
    vsm_out[...] = jnp.where(us2[...] == vs2[...], DUMP, vs2[...])


def _edge_body(u, v, e, We, be,
               Wau, Wae, Wav, ba_i,
               Wmu, Wme, Wmv, bm_i,
               Weu, Wee, Wev, ben_i,
               wm_out, wden_out, ne_out, first, last):
    uu = u[...]
    vv = v[...]
    dot = functools.partial(jnp.dot, preferred_element_type=jnp.float32)
    if first:
        ee = jax.nn.leaky_relu(dot(e[...], We[...]) + be[...])
    else:
        ee = e[...]
    att = jax.nn.leaky_relu(
        dot(uu, Wau[...]) + dot(ee, Wae[...]) + dot(vv, Wav[...]) + ba_i[...])
    msg = jax.nn.relu(
        dot(uu, Wmu[...]) + dot(ee, Wme[...]) + dot(vv, Wmv[...]) + bm_i[...])
    if not last:
        ne_out[...] = jax.nn.leaky_relu(
            dot(uu, Weu[...]) + dot(ee, Wee[...]) + dot(vv, Wev[...]) + ben_i[...])
    gmax = jnp.max(att)
    wu = jnp.exp(att - gmax)
    col = lax.broadcasted_iota(jnp.int32, (E, H), 1)
    wm_out[...] = msg * wu
    wden_out[...] = jnp.where(col == 0, wu, 0.0)


def _gru_core(num0, num1, den0, den1, h, Wih_i, bih_i, Whh_i, bhh_i,
              relu_flag):
    s = num0[...] + num1[...]
    denom = jnp.maximum(den0[:, :1] + den1[:, :1], 1e-30)
    ctx = s / denom
    dot = functools.partial(jnp.dot, preferred_element_type=jnp.float32)
    gi = dot(ctx, Wih_i[...]) + bih_i[...]
    gh = dot(h[...], Whh_i[...]) + bhh_i[...]
    r = jax.nn.sigmoid(gi[:, :H] + gh[:, :H])
    z = jax.nn.sigmoid(gi[:, H:2 * H] + gh[:, H:2 * H])
    n = jnp.tanh(gi[:, 2 * H:] + r * gh[:, 2 * H:])
    new_h = (1.0 - z) * n + z * h[...]
    if relu_flag:
        new_h = jax.nn.relu(new_h)
    return new_h


def _gru_body(num0, num1, den0, den1, h, Wih_i, bih_i, Whh_i, bhh_i,
              relu_flag, h_out):
    h_out[...] = _gru_core(num0, num1, den0, den1, h, Wih_i, bih_i,
                           Whh_i, bhh_i, relu_flag)


def _gru_readout_body(num0, num1, den0, den1, h, Wih_i, bih_i, Whh_i, bhh_i,
                      mask, mat, Wra, bra, Wro, bro, ro_out, a_out):
    hh = _gru_core(num0, num1, den0, den1, h, Wih_i, bih_i, Whh_i, bhh_i,
                   relu_flag=False)
    t = jnp.tanh(jnp.dot(hh, Wro[...], preferred_element_type=jnp.float32) + bro[...])
    alT = lax.dot_general(Wra[...], hh, (((0,), (1,)), ((), ())),
                          preferred_element_type=jnp.float32)
    alT = jax.nn.leaky_relu(alT + bra[...])
    logits = mask[...] + alT
    rmax = jnp.max(logits, axis=1, keepdims=True)
    ex = jnp.exp(logits - rmax)
    a = ex / jnp.sum(ex, axis=1, keepdims=True) * mat[...]
    a_out[...] = a
    ro_out[...] = jnp.dot(a, t, preferred_element_type=jnp.float32)


def _readout_body(h, mask, mat, Wra, bra, Wro, bro, ro_out, a_out):
    hh = h[...]
    t = jnp.tanh(jnp.dot(hh, Wro[...], preferred_element_type=jnp.float32) + bro[...])
    alT = lax.dot_general(Wra[...], hh, (((0,), (1,)), ((), ())),
                          preferred_element_type=jnp.float32)
    alT = jax.nn.leaky_relu(alT + bra[...])
    logits = mask[...] + alT
    rmax = jnp.max(logits, axis=1, keepdims=True)
    ex = jnp.exp(logits - rmax)
    a = ex / jnp.sum(ex, axis=1, keepdims=True) * mat[...]
    a_out[...] = a
    ro_out[...] = jnp.dot(a, t, preferred_element_type=jnp.float32)


def _tc_proj(nf, us2, vs2, Wn, bn):
    return pl.pallas_call(
        _proj_body,
        out_shape=[jax.ShapeDtypeStruct((N, H), jnp.float32),
                   jax.ShapeDtypeStruct((E // 128, 128), jnp.int32)],
    )(nf, us2, vs2, Wn, bn)


def _tc_edge(u, v, e, We, be, ws, first, last):
    out_shape = [jax.ShapeDtypeStruct((E, WCOL), jnp.float32),
                 jax.ShapeDtypeStruct((E, WCOL), jnp.float32),
                 jax.ShapeDtypeStruct((E, HE), jnp.float32)]

    def wrapped(u, v, e, We, be, *rest):
        args, outs = rest[:12], rest[12:]
        if last:
            outs = (*outs, None)
        _edge_body(u, v, e, We, be, *args, *outs, first=first, last=last)

    if last:
        out_shape = out_shape[:2]
    return pl.pallas_call(
        wrapped,
        out_shape=out_shape,
    )(u, v, e, We, be, *ws)


def _tc_gru(num0, num1, den0, den1, h, Wih_i, bih_i, Whh_i, bhh_i, relu_flag):
    def wrapped(n0, n1, d0, d1, hh, a, b, c, d, h_out):
        _gru_body(n0, n1, d0, d1, hh, a, b, c, d, relu_flag, h_out)

    return pl.pallas_call(
        wrapped,
        out_shape=jax.ShapeDtypeStruct((N, H), jnp.float32),
    )(num0, num1, den0, den1, h, Wih_i, bih_i, Whh_i, bhh_i)


def _tc_readout(h, mask, mat, Wra, bra, Wro, bro):
    return pl.pallas_call(
        _readout_body,
        out_shape=[jax.ShapeDtypeStruct((M, H), jnp.float32),
                   jax.ShapeDtypeStruct((M, N), jnp.float32)],
    )(h, mask, mat, Wra, bra, Wro, bro)


def _tc_gru_readout(num0, num1, den0, den1, h, Wih_i, bih_i, Whh_i, bhh_i,
                    mask, mat, Wra, bra, Wro, bro):
    return pl.pallas_call(
        _gru_readout_body,
        out_shape=[jax.ShapeDtypeStruct((M, H), jnp.float32),
                   jax.ShapeDtypeStruct((M, N), jnp.float32)],
    )(num0, num1, den0, den1, h, Wih_i, bih_i, Whh_i, bhh_i,
      mask, mat, Wra, bra, Wro, bro)


def _sc_gather(h, us2, vs2):
    mesh = plsc.VectorSubcoreMesh(core_axis_name="c", subcore_axis_name="s")

    @functools.partial(
        pl.kernel,
        out_type=[jax.ShapeDtypeStruct((E, H), jnp.float32),
                  jax.ShapeDtypeStruct((E, H), jnp.float32)],
        mesh=mesh,
        scratch_types=[pltpu.VMEM((IPW, 128), jnp.int32),
                       pltpu.VMEM((IPW, 128), jnp.int32),
                       pltpu.VMEM((EPT, H), jnp.float32),
                       pltpu.VMEM((EPT, H), jnp.float32),
                       pltpu.SemaphoreType.DMA,
                       pltpu.SemaphoreType.DMA],
    )
    def k(h_hbm, us_hbm, vs_hbm, u_out, v_out, idxu, idxv, rowsu, rowsv,
          semu, semv):
        cid = lax.axis_index("c")
        sid = lax.axis_index("s")
        wid = sid * NC + cid
        base = wid * EPT
        cps = [pltpu.async_copy(us_hbm.at[pl.ds(wid * IPW, IPW)], idxu, semu),
               pltpu.async_copy(vs_hbm.at[pl.ds(wid * IPW, IPW)], idxv, semu)]
        for cp in cps:
            cp.wait()
        cps = []
        for j in range(IPW):
            cps.append(pltpu.async_copy(
                h_hbm.at[idxu.at[j]], rowsu.at[pl.ds(j * 128, 128)], semu))
            cps.append(pltpu.async_copy(
                h_hbm.at[idxv.at[j]], rowsv.at[pl.ds(j * 128, 128)], semv))
        for cp in cps:
            cp.wait()
        cps = [pltpu.async_copy(rowsu, u_out.at[pl.ds(base, EPT)], semu),
               pltpu.async_copy(rowsv, v_out.at[pl.ds(base, EPT)], semv)]
        for cp in cps:
            cp.wait()

    return k(h, us2, vs2)


def _sc_scatter(wm, wden, us2, vs2, zer):
    mesh = plsc.VectorSubcoreMesh(core_axis_name="c", subcore_axis_name="s")

    @functools.partial(
        pl.kernel,
        out_type=[jax.ShapeDtypeStruct((N, WCOL), jnp.float32),
                  jax.ShapeDtypeStruct((N, WCOL), jnp.float32),
                  jax.ShapeDtypeStruct((N, WCOL), jnp.float32),
                  jax.ShapeDtypeStruct((N, WCOL), jnp.float32)],
        mesh=mesh,
        scratch_types=[pltpu.VMEM((IPW, 128), jnp.int32),
                       pltpu.VMEM((IPW, 128), jnp.int32),
                       pltpu.VMEM((EPT, WCOL), jnp.float32),
                       pltpu.VMEM((EPT, WCOL), jnp.float32),
                       pltpu.VMEM_SHARED((NACC, WCOL), jnp.float32),
                       pltpu.VMEM_SHARED((NACC, WCOL), jnp.float32),
                       pltpu.SemaphoreType.DMA],
    )
    def k(wm_hbm, wden_hbm, us_hbm, vs_hbm, zer_hbm, num_out0, num_out1,
          den_out0, den_out1, idxu, idxv, rows_m, rows_d, acc_n, acc_d, sem):
        cid = lax.axis_index("c")
        sid = lax.axis_index("s")
        wid = sid * NC + cid
        base = wid * EPT
        cps = [
            pltpu.async_copy(zer_hbm.at[pl.ds(sid * NROWS, NROWS)],
                             acc_n.at[pl.ds(sid * NROWS, NROWS)], sem),
            pltpu.async_copy(zer_hbm.at[pl.ds(sid * NROWS, NROWS)],
                             acc_d.at[pl.ds(sid * NROWS, NROWS)], sem),
            pltpu.async_copy(us_hbm.at[pl.ds(wid * IPW, IPW)], idxu, sem),
            pltpu.async_copy(vs_hbm.at[pl.ds(wid * IPW, IPW)], idxv, sem),
            pltpu.async_copy(wm_hbm.at[pl.ds(base, EPT)], rows_m, sem),
            pltpu.async_copy(wden_hbm.at[pl.ds(base, EPT)], rows_d, sem),
        ]
        for cp in cps:
            cp.wait()
        plsc.subcore_barrier()
        cps = []
        for j in range(IPW):
            cps.append(pltpu.async_copy(rows_m.at[pl.ds(j * 128, 128)],
                                        acc_n.at[idxu.at[j]], sem, add=True))
            cps.append(pltpu.async_copy(rows_m.at[pl.ds(j * 128, 128)],
                                        acc_n.at[idxv.at[j]], sem, add=True))
            cps.append(pltpu.async_copy(rows_d.at[pl.ds(j * 128, 128)],
                                        acc_d.at[idxu.at[j]], sem, add=True))
            cps.append(pltpu.async_copy(rows_d.at[pl.ds(j * 128, 128)],
                                        acc_d.at[idxv.at[j]], sem, add=True))
        for cp in cps:
            cp.wait()
        plsc.subcore_barrier()

        @pl.when(cid == 0)
        def _():
            pltpu.sync_copy(acc_n.at[pl.ds(sid * NROWS, NROWS)],
                            num_out0.at[pl.ds(sid * NROWS, NROWS)])
            pltpu.sync_copy(acc_d.at[pl.ds(sid * NROWS, NROWS)],
                            den_out0.at[pl.ds(sid * NROWS, NROWS)])

        @pl.when(cid == 1)
        def _():
            pltpu.sync_copy(acc_n.at[pl.ds(sid * NROWS, NROWS)],
                            num_out1.at[pl.ds(sid * NROWS, NROWS)])
            pltpu.sync_copy(acc_d.at[pl.ds(sid * NROWS, NROWS)],
                            den_out1.at[pl.ds(sid * NROWS, NROWS)])

    return k(wm, wden, us2, vs2, zer)


def kernel(node_features, edge_features, us, vs, mol_node_matrix, mol_node_mask,
           node_edge_matrix, node_edge_mask, global_mask, W_n, b_n, W_e, b_e,
           Wm, bm, Wa, ba, Wen, ben, Wih, bih, Whh, bhh, Wra, bra, Wro, bro):
    us2 = us.reshape(E // 128, 128)
    vs2 = vs.reshape(E // 128, 128)
    zer = jnp.zeros((N, WCOL), jnp.float32)

    h, vsm = _tc_proj(node_features, us2, vs2, W_n, b_n.reshape(1, H))
    e = edge_features
    We, be = W_e, b_e.reshape(1, HE)

    for i in range(L):
        first, last = (i == 0), (i == L - 1)
        ws = (Wa[i][:H], Wa[i][H:H + HE], Wa[i][H + HE:], ba[i].reshape(1, 1),
              Wm[i][:H], Wm[i][H:H + HE], Wm[i][H + HE:], bm[i].reshape(1, C),
              Wen[i][:H], Wen[i][H:H + HE], Wen[i][H + HE:], ben[i].reshape(1, HE))
        u, v = _sc_gather(h, us2, vs2)
        outs = _tc_edge(u, v, e, We, be, ws, first, last)
        if last:
            wm, wden = outs
        else:
            wm, wden, e = outs
            We = jnp.eye(HE, dtype=jnp.float32)
        num0, num1, den0, den1 = _sc_scatter(wm, wden, us2, vsm, zer)
        if last:
            readout, a = _tc_gru_readout(
                num0, num1, den0, den1, h,
                Wih[i], bih[i].reshape(1, 3 * H),
                Whh[i], bhh[i].reshape(1, 3 * H),
                mol_node_mask, mol_node_matrix,
                Wra, bra.reshape(1, 1), Wro, bro.reshape(1, H))
        else:
            h = _tc_gru(num0, num1, den0, den1, h,
                        Wih[i], bih[i].reshape(1, 3 * H),
                        Whh[i], bhh[i].reshape(1, 3 * H), relu_flag=True)

    return readout, a
